# Initial kernel scaffold; baseline (speedup 1.0000x reference)
import jax
import jax.numpy as jnp
from jax.experimental import pallas as pl


def kernel(w1, b1, g1, be1, w2, b2, g2, be2, w3, b3, g3, be3, x):
    raise NotImplementedError("write your pallas kernel here")


# speedup vs baseline: 1.1069x; 1.1069x over previous
import functools

import jax
import jax.numpy as jnp
from jax.experimental import pallas as pl
from jax.experimental.pallas import tpu as pltpu

_EPS = 1e-5


def _lshift(a, s):
    if s == 0:
        return a
    pad = jnp.zeros((a.shape[0], abs(s)), a.dtype)
    if s > 0:
        return jnp.concatenate([a[:, s:], pad], axis=1)
    return jnp.concatenate([pad, a[:, :s]], axis=1)


def _stats(y):
    s = jnp.sum(y, axis=1, keepdims=True)
    ss = jnp.sum(y * y, axis=1, keepdims=True)
    return jnp.concatenate([s, ss], axis=1)[None]


def _fold(st_ref, g_ref, be_ref, m):
    tot = jnp.sum(st_ref[...], axis=0)
    mean = tot[:, 0:1] / m
    var = tot[:, 1:2] / m - mean * mean
    sc = g_ref[...] * jax.lax.rsqrt(var + _EPS)
    sh = be_ref[...] - mean * sc
    return sc, sh


def _s1_kernel(x_ref, w_ref, y_ref, st_ref):
    x = x_ref[0].astype(jnp.bfloat16)
    y = jnp.dot(w_ref[...], x, preferred_element_type=jnp.float32)
    y_ref[0] = y.astype(jnp.bfloat16)
    st_ref[...] = _stats(y)


def _s2_kernel(y1_ref, st_ref, g_ref, be_ref, w_ref, y_ref, st2_ref, *, m, width):
    sc, sh = _fold(st_ref, g_ref, be_ref, m)
    a = jnp.maximum(y1_ref[0].astype(jnp.float32) * sc + sh, 0.0)
    a = a.astype(jnp.bfloat16)
    hw = a.shape[1]
    col = jax.lax.broadcasted_iota(jnp.int32, (1, hw), 1) % width
    a_l = jnp.where(col > 0, _lshift(a, -1), jnp.bfloat16(0))
    a_r = jnp.where(col < width - 1, _lshift(a, 1), jnp.bfloat16(0))
    a3 = jnp.concatenate([a_l, a, a_r], axis=0)
    taps = jnp.concatenate(
        [_lshift(a3, -width), a3, _lshift(a3, width)], axis=0)
    y = jnp.dot(w_ref[...], taps, preferred_element_type=jnp.float32)
    y_ref[0] = y.astype(jnp.bfloat16)
    st2_ref[...] = _stats(y)


def _s3_kernel(y2_ref, st_ref, g_ref, be_ref, w_ref, y_ref, st3_ref, *, m):
    sc, sh = _fold(st_ref, g_ref, be_ref, m)
    a = jnp.maximum(y2_ref[0].astype(jnp.float32) * sc + sh, 0.0)
    y = jnp.dot(w_ref[...], a.astype(jnp.bfloat16),
                preferred_element_type=jnp.float32)
    y_ref[0] = y.astype(jnp.bfloat16)
    st3_ref[...] = _stats(y)


def _s4_kernel(y3_ref, st_ref, g_ref, be_ref, x_ref, o_ref, *, m):
    sc, sh = _fold(st_ref, g_ref, be_ref, m)
    o_ref[0] = jnp.maximum(
        y3_ref[0].astype(jnp.float32) * sc + sh + x_ref[0], 0.0)


def kernel(w1, b1, g1, be1, w2, b2, g2, be2, w3, b3, g3, be3, x):
    N, C, H, W = x.shape
    HW = H * W
    M = N * HW
    f32, bf16 = jnp.float32, jnp.bfloat16

    x3 = x.reshape(N, C, HW)
    w1t = w1.T.astype(bf16)
    w2t = w2.reshape(9 * C, C).T.astype(bf16)
    w3t = w3.T.astype(bf16)
    g1c, be1c = g1.reshape(C, 1), be1.reshape(C, 1)
    g2c, be2c = g2.reshape(C, 1), be2.reshape(C, 1)
    g3c, be3c = g3.reshape(C, 1), be3.reshape(C, 1)

    par = pltpu.CompilerParams(dimension_semantics=("parallel",))
    img = pl.BlockSpec((1, C, HW), lambda n: (n, 0, 0))
    stat_o = pl.BlockSpec((1, C, 2), lambda n: (n, 0, 0))
    stat_i = pl.BlockSpec((N, C, 2), lambda n: (0, 0, 0))
    vec = pl.BlockSpec((C, 1), lambda n: (0, 0))

    def mat(shape):
        return pl.BlockSpec(shape, lambda n: (0, 0))

    act_bf = jax.ShapeDtypeStruct((N, C, HW), bf16)
    st_f32 = jax.ShapeDtypeStruct((N, C, 2), f32)

    y1, st1 = pl.pallas_call(
        _s1_kernel,
        grid=(N,),
        in_specs=[img, mat((C, C))],
        out_specs=[img, stat_o],
        out_shape=[act_bf, st_f32],
        compiler_params=par,
    )(x3, w1t)

    y2, st2 = pl.pallas_call(
        functools.partial(_s2_kernel, m=M, width=W),
        grid=(N,),
        in_specs=[img, stat_i, vec, vec, mat((C, 9 * C))],
        out_specs=[img, stat_o],
        out_shape=[act_bf, st_f32],
        compiler_params=par,
    )(y1, st1, g1c, be1c, w2t)

    y3, st3 = pl.pallas_call(
        functools.partial(_s3_kernel, m=M),
        grid=(N,),
        in_specs=[img, stat_i, vec, vec, mat((C, C))],
        out_specs=[img, stat_o],
        out_shape=[act_bf, st_f32],
        compiler_params=par,
    )(y2, st2, g2c, be2c, w3t)

    out = pl.pallas_call(
        functools.partial(_s4_kernel, m=M),
        grid=(N,),
        in_specs=[img, stat_i, vec, vec, img],
        out_specs=img,
        out_shape=jax.ShapeDtypeStruct((N, C, HW), f32),
        compiler_params=par,
    )(y3, st3, g3c, be3c, x3)

    return out.reshape(N, C, H, W)


# speedup vs baseline: 1.4669x; 1.3252x over previous
import functools

import jax
import jax.numpy as jnp
from jax.experimental import pallas as pl
from jax.experimental.pallas import tpu as pltpu

_EPS = 1e-5
_CONTRACT0 = (((0,), (0,)), ((), ()))


def _lshift(a, s):
    if s == 0:
        return a
    pad = jnp.zeros((a.shape[0], abs(s)), a.dtype)
    if s > 0:
        return jnp.concatenate([a[:, s:], pad], axis=1)
    return jnp.concatenate([pad, a[:, :s]], axis=1)


def _stats(y, acc):
    s = jnp.sum(y, axis=1, keepdims=True)
    ss = jnp.sum(y * y, axis=1, keepdims=True)
    st = jnp.concatenate([s, ss], axis=1)
    return st if acc is None else acc + st


def _fold(st_ref, g_ref, be_ref, m):
    tot = jnp.sum(st_ref[...], axis=0)
    mean = tot[:, 0:1] / m
    var = tot[:, 1:2] / m - mean * mean
    sc = g_ref[...] * jax.lax.rsqrt(var + _EPS)
    sh = be_ref[...] - mean * sc
    return sc, sh


def _s1_kernel(x_ref, w_ref, y_ref, st_ref):
    acc = None
    for i in range(x_ref.shape[0]):
        x = x_ref[i].astype(jnp.bfloat16)
        y = jax.lax.dot_general(w_ref[...], x, _CONTRACT0,
                                preferred_element_type=jnp.float32)
        y_ref[i] = y.astype(jnp.bfloat16)
        acc = _stats(y, acc)
    st_ref[...] = acc[None]


def _s2_kernel(y1_ref, st_ref, g_ref, be_ref, w_ref, y_ref, st2_ref, *, m, width):
    sc, sh = _fold(st_ref, g_ref, be_ref, m)
    hw = y1_ref.shape[2]
    col = jax.lax.broadcasted_iota(jnp.int32, (1, hw), 1) % width
    acc = None
    for i in range(y1_ref.shape[0]):
        a = jnp.maximum(y1_ref[i].astype(jnp.float32) * sc + sh, 0.0)
        a = a.astype(jnp.bfloat16)
        a_l = jnp.where(col > 0, _lshift(a, -1), jnp.bfloat16(0))
        a_r = jnp.where(col < width - 1, _lshift(a, 1), jnp.bfloat16(0))
        a3 = jnp.concatenate([a_l, a, a_r], axis=0)
        taps = jnp.concatenate(
            [_lshift(a3, -width), a3, _lshift(a3, width)], axis=0)
        y = jax.lax.dot_general(w_ref[...], taps, _CONTRACT0,
                                preferred_element_type=jnp.float32)
        y_ref[i] = y.astype(jnp.bfloat16)
        acc = _stats(y, acc)
    st2_ref[...] = acc[None]


def _s3_kernel(y2_ref, st_ref, g_ref, be_ref, w_ref, y_ref, st3_ref, *, m):
    sc, sh = _fold(st_ref, g_ref, be_ref, m)
    acc = None
    for i in range(y2_ref.shape[0]):
        a = jnp.maximum(y2_ref[i].astype(jnp.float32) * sc + sh, 0.0)
        y = jax.lax.dot_general(w_ref[...], a.astype(jnp.bfloat16), _CONTRACT0,
                                preferred_element_type=jnp.float32)
        y_ref[i] = y.astype(jnp.bfloat16)
        acc = _stats(y, acc)
    st3_ref[...] = acc[None]


def _s4_kernel(y3_ref, st_ref, g_ref, be_ref, x_ref, o_ref, *, m):
    sc, sh = _fold(st_ref, g_ref, be_ref, m)
    for i in range(y3_ref.shape[0]):
        o_ref[i] = jnp.maximum(
            y3_ref[i].astype(jnp.float32) * sc + sh + x_ref[i], 0.0)


def kernel(w1, b1, g1, be1, w2, b2, g2, be2, w3, b3, g3, be3, x):
    N, C, H, W = x.shape
    HW = H * W
    M = N * HW
    f32, bf16 = jnp.float32, jnp.bfloat16
    G = 4 if N % 4 == 0 else 1
    nst = N // G

    x3 = x.reshape(N, C, HW)
    w1b = w1.astype(bf16)
    w2b = w2.reshape(9 * C, C).astype(bf16)
    w3b = w3.astype(bf16)
    g1c, be1c = g1.reshape(C, 1), be1.reshape(C, 1)
    g2c, be2c = g2.reshape(C, 1), be2.reshape(C, 1)
    g3c, be3c = g3.reshape(C, 1), be3.reshape(C, 1)

    par = pltpu.CompilerParams(dimension_semantics=("parallel",))
    img = pl.BlockSpec((G, C, HW), lambda n: (n, 0, 0))
    stat_o = pl.BlockSpec((1, C, 2), lambda n: (n, 0, 0))
    stat_i = pl.BlockSpec((nst, C, 2), lambda n: (0, 0, 0))
    vec = pl.BlockSpec((C, 1), lambda n: (0, 0))

    def mat(shape):
        return pl.BlockSpec(shape, lambda n: (0, 0))

    act_bf = jax.ShapeDtypeStruct((N, C, HW), bf16)
    st_f32 = jax.ShapeDtypeStruct((nst, C, 2), f32)

    y1, st1 = pl.pallas_call(
        _s1_kernel,
        grid=(nst,),
        in_specs=[img, mat((C, C))],
        out_specs=[img, stat_o],
        out_shape=[act_bf, st_f32],
        compiler_params=par,
    )(x3, w1b)

    y2, st2 = pl.pallas_call(
        functools.partial(_s2_kernel, m=M, width=W),
        grid=(nst,),
        in_specs=[img, stat_i, vec, vec, mat((9 * C, C))],
        out_specs=[img, stat_o],
        out_shape=[act_bf, st_f32],
        compiler_params=par,
    )(y1, st1, g1c, be1c, w2b)

    y3, st3 = pl.pallas_call(
        functools.partial(_s3_kernel, m=M),
        grid=(nst,),
        in_specs=[img, stat_i, vec, vec, mat((C, C))],
        out_specs=[img, stat_o],
        out_shape=[act_bf, st_f32],
        compiler_params=par,
    )(y2, st2, g2c, be2c, w3b)

    out = pl.pallas_call(
        functools.partial(_s4_kernel, m=M),
        grid=(nst,),
        in_specs=[img, stat_i, vec, vec, img],
        out_specs=img,
        out_shape=jax.ShapeDtypeStruct((N, C, HW), f32),
        compiler_params=par,
    )(y3, st3, g3c, be3c, x3)

    return out.reshape(N, C, H, W)
